# baseline (speedup 1.0000x reference)
import functools

import jax
import jax.numpy as jnp
from jax import lax
from jax.experimental import pallas as pl
from jax.experimental.pallas import tpu as pltpu
from jax.experimental.pallas import tpu_sc as plsc

N = 10000
E = 320000
D = 128
G = 128
NUM_CLASSES = 7

NC, NS = 2, 16
NTILES = NC * NS
CHUNK = 112
NSLOT = 3
NISLOT = 6
NCH0 = 90
NCH1 = 90
NCH = max(NCH0, NCH1)
E_PAD = NS * (NCH0 + NCH1) * CHUNK
NPAD = 10240
NACC = 10112
SLAB = NACC // NS

BLK = 256
NBLK = NPAD // BLK


@functools.cache
def _make_sc_agg():
    @functools.partial(
        pl.kernel,
        out_type=jax.ShapeDtypeStruct((NC, NPAD, D), jnp.float32),
        mesh=plsc.VectorSubcoreMesh(
            core_axis_name="c", subcore_axis_name="s",
            num_cores=NC, num_subcores=NS),
        scratch_types=[
            [pltpu.VMEM((8, CHUNK), jnp.int32) for _ in range(NISLOT)],
            [pltpu.VMEM((CHUNK, D), jnp.float32) for _ in range(NSLOT)],
            pltpu.VMEM_SHARED((NACC, D), jnp.float32),
            [pltpu.SemaphoreType.DMA for _ in range(NISLOT)],
            [pltpu.SemaphoreType.DMA for _ in range(NSLOT)],
            [pltpu.SemaphoreType.DMA for _ in range(NSLOT)],
        ],
    )
    def _sc_agg(x_hbm, idx_hbm, zeros_hbm, out_hbm,
                islots, bufs, acc, isems, gsems, ssems):
        c = lax.axis_index("c")
        s = lax.axis_index("s")
        wid = c * NS + s
        nch = jnp.where(c == 0, NCH0, NCH1)

        def idx_start(j, i):
            pltpu.async_copy(idx_hbm.at[wid, j], islots[i], isems[i])

        def idx_wait(i):
            pltpu.make_async_copy(
                idx_hbm.at[wid, 0], islots[i], isems[i]).wait()

        def gather_start(i, b):
            pltpu.async_copy(x_hbm.at[islots[i].at[0]], bufs[b], gsems[b])

        def gather_wait(i, b):
            pltpu.make_async_copy(
                x_hbm.at[islots[i].at[0]], bufs[b], gsems[b]).wait()

        def scatter_start(i, b):
            pltpu.async_copy(bufs[b], acc.at[islots[i].at[1]], ssems[b],
                             add=True)

        def scatter_wait(i, b):
            pltpu.make_async_copy(
                bufs[b], acc.at[islots[i].at[1]], ssems[b]).wait()

        for i in range(NISLOT - 1):
            idx_start(i, i)
        pltpu.sync_copy(zeros_hbm, acc.at[pl.ds(s * SLAB, SLAB)])
        plsc.subcore_barrier()
        idx_wait(0)
        gather_start(0, 0)
        idx_wait(1)
        gather_start(1, 1)

        def body(t, carry):
            for b in range(NISLOT):
                j = t * NISLOT + b
                bb = b % NSLOT
                gather_wait(b, bb)
                scatter_start(b, bb)

                @pl.when(j >= 1)
                def _():
                    scatter_wait((b + 5) % NISLOT, (bb + 2) % NSLOT)

                @pl.when(j + 2 < nch)
                def _():
                    idx_wait((b + 2) % NISLOT)
                    gather_start((b + 2) % NISLOT, (bb + 2) % NSLOT)

                @pl.when(j + 5 < nch)
                def _():
                    idx_start(j + 5, (b + 5) % NISLOT)
            return carry

        lax.fori_loop(0, nch // NISLOT, body, 0)
        scatter_wait(NISLOT - 1, NSLOT - 1)
        plsc.subcore_barrier()
        pltpu.sync_copy(acc.at[pl.ds(s * SLAB, SLAB)],
                        out_hbm.at[c, pl.ds(s * SLAB, SLAB)])

    return _sc_agg


def _mlp_body(x_ref, a_ref, wa_ref, ba_ref, wb_ref, bb_ref, o_ref):
    i = pl.program_id(0)
    t = x_ref[...] + a_ref[0] + a_ref[1]
    u = jnp.maximum(
        jnp.dot(t, wa_ref[...], preferred_element_type=jnp.float32)
        + ba_ref[...], 0.0)
    v = jnp.maximum(
        jnp.dot(u, wb_ref[...], preferred_element_type=jnp.float32)
        + bb_ref[...], 0.0)
    rows = i * BLK + lax.broadcasted_iota(jnp.int32, (BLK, D), 0)
    o_ref[...] = jnp.where(rows < N, v, 0.0)


def _mlp(x, a, wa, ba, wb, bb):
    row_spec = pl.BlockSpec((BLK, D), lambda i: (i, 0))
    agg_spec = pl.BlockSpec((NC, BLK, D), lambda i: (0, i, 0))
    full = pl.BlockSpec((D, D), lambda i: (0, 0))
    bias = pl.BlockSpec((1, D), lambda i: (0, 0))
    return pl.pallas_call(
        _mlp_body,
        grid=(NBLK,),
        in_specs=[row_spec, agg_spec, full, bias, full, bias],
        out_specs=row_spec,
        out_shape=jax.ShapeDtypeStruct((NPAD, D), jnp.float32),
    )(x, a, wa, ba.reshape(1, D), wb, bb.reshape(1, D))


def _head_body(h_ref, b_ref, l1w_ref, l1b_ref, bng_ref, bnb_ref, bnm_ref,
               bnv_ref, l2w_ref, l2b_ref, o_ref, sums, cnts):
    i = pl.program_id(0)

    @pl.when(i == 0)
    def _():
        sums[...] = jnp.zeros_like(sums)
        cnts[...] = jnp.zeros_like(cnts)

    b = b_ref[0, 0, :]
    gids = lax.broadcasted_iota(jnp.int32, (G, BLK), 0)
    mask = (b[None, :] == gids).astype(jnp.float32)
    sums[...] += jnp.dot(mask, h_ref[...], preferred_element_type=jnp.float32)
    cnts[...] += jnp.broadcast_to(
        jnp.sum(mask, axis=1, keepdims=True), (G, D))

    @pl.when(i == NBLK - 1)
    def _():
        pooled = sums[...] / jnp.maximum(cnts[...], 1.0)
        g = jnp.dot(pooled, l1w_ref[...],
                    preferred_element_type=jnp.float32) + l1b_ref[...]
        g = (g - bnm_ref[...]) * jax.lax.rsqrt(bnv_ref[...] + 1e-5)
        g = g * bng_ref[...] + bnb_ref[...]
        g = jnp.maximum(g, 0.0)
        o_ref[...] = jnp.dot(g, l2w_ref[...],
                             preferred_element_type=jnp.float32) + l2b_ref[...]


def _head(h, batch3, l1w, l1b, bng, bnb, bnm, bnv, l2w, l2b):
    H2 = 2 * D
    return pl.pallas_call(
        _head_body,
        grid=(NBLK,),
        in_specs=[
            pl.BlockSpec((BLK, D), lambda i: (i, 0)),
            pl.BlockSpec((1, 1, BLK), lambda i: (i, 0, 0)),
            pl.BlockSpec((D, H2), lambda i: (0, 0)),
            pl.BlockSpec((1, H2), lambda i: (0, 0)),
            pl.BlockSpec((1, H2), lambda i: (0, 0)),
            pl.BlockSpec((1, H2), lambda i: (0, 0)),
            pl.BlockSpec((1, H2), lambda i: (0, 0)),
            pl.BlockSpec((1, H2), lambda i: (0, 0)),
            pl.BlockSpec((H2, D), lambda i: (0, 0)),
            pl.BlockSpec((1, D), lambda i: (0, 0)),
        ],
        out_specs=pl.BlockSpec((G, D), lambda i: (0, 0)),
        out_shape=jax.ShapeDtypeStruct((G, D), jnp.float32),
        scratch_shapes=[
            pltpu.VMEM((G, D), jnp.float32),
            pltpu.VMEM((G, D), jnp.float32),
        ],
    )(h, batch3, l1w, l1b, bng, bnb, bnm, bnv, l2w, l2b)


def kernel(x, edge_index, batch, w1a, b1a, w1b, b1b, w2a, b2a, w2b, b2b,
           w3a, b3a, w3b, b3b, lin1_w, lin1_b, bn_g, bn_b, bn_rm, bn_rv,
           lin2_w, lin2_b):
    x = x.astype(jnp.float32)
    src = edge_index[0].astype(jnp.int32)
    dst = edge_index[1].astype(jnp.int32)
    cap0 = NS * NCH0 * CHUNK
    src_p = jnp.concatenate([src, jnp.full((E_PAD - E,), N, jnp.int32)])
    dst_p = jnp.concatenate([dst, jnp.zeros((E_PAD - E,), jnp.int32)])

    def tile_layout(v):
        v0 = v[:cap0].reshape(NS, NCH0, 1, CHUNK)
        v1 = v[cap0:].reshape(NS, NCH1, 1, CHUNK)
        v0 = jnp.pad(v0, ((0, 0), (0, NCH - NCH0), (0, 0), (0, 0)),
                     constant_values=N)
        v1 = jnp.pad(v1, ((0, 0), (0, NCH - NCH1), (0, 0), (0, 0)),
                     constant_values=N)
        return jnp.concatenate([v0, v1], axis=0)

    idx4 = jnp.concatenate([tile_layout(src_p),
                            jnp.where(tile_layout(dst_p) == N, 0,
                                      tile_layout(dst_p)),
                            jnp.zeros((NTILES, NCH, 6, CHUNK), jnp.int32)],
                           axis=2)
    zeros = jnp.zeros((SLAB, D), jnp.float32)

    x_pad = jnp.zeros((NPAD, D), jnp.float32).at[:N].set(x)
    batch3 = jnp.concatenate([
        batch.astype(jnp.int32), jnp.full((NPAD - N,), G, jnp.int32)
    ]).reshape(NBLK, 1, BLK)

    h = x_pad
    for wa, ba, wb, bb in ((w1a, b1a, w1b, b1b),
                           (w2a, b2a, w2b, b2b),
                           (w3a, b3a, w3b, b3b)):
        agg = _make_sc_agg()(h, idx4, zeros)
        h = _mlp(h, agg, wa, ba, wb, bb)

    H2 = 2 * D
    l2w_pad = jnp.zeros((H2, D), jnp.float32).at[:, :NUM_CLASSES].set(lin2_w)
    l2b_pad = jnp.zeros((1, D), jnp.float32).at[0, :NUM_CLASSES].set(lin2_b)
    out = _head(h, batch3, lin1_w, lin1_b.reshape(1, H2),
                bn_g.reshape(1, H2), bn_b.reshape(1, H2),
                bn_rm.reshape(1, H2), bn_rv.reshape(1, H2),
                l2w_pad, l2b_pad)
    return out[:, :NUM_CLASSES]


# speedup vs baseline: 1.0984x; 1.0984x over previous
import functools

import jax
import jax.numpy as jnp
from jax import lax
from jax.experimental import pallas as pl
from jax.experimental.pallas import tpu as pltpu
from jax.experimental.pallas import tpu_sc as plsc

N = 10000
E = 320000
D = 128
G = 128
NUM_CLASSES = 7

NC, NS = 2, 16
NTILES = NC * NS
CHUNK = 112
NSLOT = 3
NISLOT = 6
NCH0 = 156
NCH1 = 24
NCH = max(NCH0, NCH1)
E_PAD = NS * (NCH0 + NCH1) * CHUNK
NPAD = 10240
NACC = 10112
SLAB = NACC // NS

BLK = 256
NBLK = NPAD // BLK


@functools.cache
def _make_sc_agg():
    @functools.partial(
        pl.kernel,
        out_type=jax.ShapeDtypeStruct((NC, NPAD, D), jnp.float32),
        mesh=plsc.VectorSubcoreMesh(
            core_axis_name="c", subcore_axis_name="s",
            num_cores=NC, num_subcores=NS),
        scratch_types=[
            [pltpu.VMEM((8, CHUNK), jnp.int32) for _ in range(NISLOT)],
            [pltpu.VMEM((CHUNK, D), jnp.float32) for _ in range(NSLOT)],
            pltpu.VMEM_SHARED((NACC, D), jnp.float32),
            [pltpu.SemaphoreType.DMA for _ in range(NISLOT)],
            [pltpu.SemaphoreType.DMA for _ in range(NSLOT)],
            [pltpu.SemaphoreType.DMA for _ in range(NSLOT)],
        ],
    )
    def _sc_agg(x_hbm, idx_hbm, zeros_hbm, out_hbm,
                islots, bufs, acc, isems, gsems, ssems):
        c = lax.axis_index("c")
        s = lax.axis_index("s")
        wid = c * NS + s
        nch = jnp.where(c == 0, NCH0, NCH1)

        def idx_start(j, i):
            pltpu.async_copy(idx_hbm.at[wid, j], islots[i], isems[i])

        def idx_wait(i):
            pltpu.make_async_copy(
                idx_hbm.at[wid, 0], islots[i], isems[i]).wait()

        def gather_start(i, b):
            pltpu.async_copy(x_hbm.at[islots[i].at[0]], bufs[b], gsems[b])

        def gather_wait(i, b):
            pltpu.make_async_copy(
                x_hbm.at[islots[i].at[0]], bufs[b], gsems[b]).wait()

        def scatter_start(i, b):
            pltpu.async_copy(bufs[b], acc.at[islots[i].at[1]], ssems[b],
                             add=True)

        def scatter_wait(i, b):
            pltpu.make_async_copy(
                bufs[b], acc.at[islots[i].at[1]], ssems[b]).wait()

        for i in range(NISLOT - 1):
            idx_start(i, i)
        pltpu.sync_copy(zeros_hbm, acc.at[pl.ds(s * SLAB, SLAB)])
        plsc.subcore_barrier()
        idx_wait(0)
        gather_start(0, 0)
        idx_wait(1)
        gather_start(1, 1)

        def body(t, carry):
            for b in range(NISLOT):
                j = t * NISLOT + b
                bb = b % NSLOT
                gather_wait(b, bb)
                scatter_start(b, bb)

                @pl.when(j >= 1)
                def _():
                    scatter_wait((b + 5) % NISLOT, (bb + 2) % NSLOT)

                @pl.when(j + 2 < nch)
                def _():
                    idx_wait((b + 2) % NISLOT)
                    gather_start((b + 2) % NISLOT, (bb + 2) % NSLOT)

                @pl.when(j + 5 < nch)
                def _():
                    idx_start(j + 5, (b + 5) % NISLOT)
            return carry

        lax.fori_loop(0, nch // NISLOT, body, 0)
        scatter_wait(NISLOT - 1, NSLOT - 1)
        plsc.subcore_barrier()
        pltpu.sync_copy(acc.at[pl.ds(s * SLAB, SLAB)],
                        out_hbm.at[c, pl.ds(s * SLAB, SLAB)])

    return _sc_agg


def _mlp_body(x_ref, a_ref, wa_ref, ba_ref, wb_ref, bb_ref, o_ref):
    i = pl.program_id(0)
    t = x_ref[...] + a_ref[0] + a_ref[1]
    u = jnp.maximum(
        jnp.dot(t, wa_ref[...], preferred_element_type=jnp.float32)
        + ba_ref[...], 0.0)
    v = jnp.maximum(
        jnp.dot(u, wb_ref[...], preferred_element_type=jnp.float32)
        + bb_ref[...], 0.0)
    rows = i * BLK + lax.broadcasted_iota(jnp.int32, (BLK, D), 0)
    o_ref[...] = jnp.where(rows < N, v, 0.0)


def _mlp(x, a, wa, ba, wb, bb):
    row_spec = pl.BlockSpec((BLK, D), lambda i: (i, 0))
    agg_spec = pl.BlockSpec((NC, BLK, D), lambda i: (0, i, 0))
    full = pl.BlockSpec((D, D), lambda i: (0, 0))
    bias = pl.BlockSpec((1, D), lambda i: (0, 0))
    return pl.pallas_call(
        _mlp_body,
        grid=(NBLK,),
        in_specs=[row_spec, agg_spec, full, bias, full, bias],
        out_specs=row_spec,
        out_shape=jax.ShapeDtypeStruct((NPAD, D), jnp.float32),
    )(x, a, wa, ba.reshape(1, D), wb, bb.reshape(1, D))


def _mlp_head_body(x_ref, a_ref, wa_ref, ba_ref, wb_ref, bb_ref, b_ref,
                   l1w_ref, l1b_ref, bng_ref, bnb_ref, bnm_ref, bnv_ref,
                   l2w_ref, l2b_ref, o_ref, sums, cnts):
    i = pl.program_id(0)

    @pl.when(i == 0)
    def _():
        sums[...] = jnp.zeros_like(sums)
        cnts[...] = jnp.zeros_like(cnts)

    t = x_ref[...] + a_ref[0] + a_ref[1]
    u = jnp.maximum(
        jnp.dot(t, wa_ref[...], preferred_element_type=jnp.float32)
        + ba_ref[...], 0.0)
    v = jnp.maximum(
        jnp.dot(u, wb_ref[...], preferred_element_type=jnp.float32)
        + bb_ref[...], 0.0)
    rows = i * BLK + lax.broadcasted_iota(jnp.int32, (BLK, D), 0)
    v = jnp.where(rows < N, v, 0.0)

    b = b_ref[0, 0, :]
    gids = lax.broadcasted_iota(jnp.int32, (G, BLK), 0)
    mask = (b[None, :] == gids).astype(jnp.float32)
    sums[...] += jnp.dot(mask, v, preferred_element_type=jnp.float32)
    cnts[...] += jnp.broadcast_to(
        jnp.sum(mask, axis=1, keepdims=True), (G, D))

    @pl.when(i == NBLK - 1)
    def _():
        pooled = sums[...] / jnp.maximum(cnts[...], 1.0)
        g = jnp.dot(pooled, l1w_ref[...],
                    preferred_element_type=jnp.float32) + l1b_ref[...]
        g = (g - bnm_ref[...]) * jax.lax.rsqrt(bnv_ref[...] + 1e-5)
        g = g * bng_ref[...] + bnb_ref[...]
        g = jnp.maximum(g, 0.0)
        o_ref[...] = jnp.dot(g, l2w_ref[...],
                             preferred_element_type=jnp.float32) + l2b_ref[...]


def _mlp_head(h, agg, wa, ba, wb, bb, batch3,
              l1w, l1b, bng, bnb, bnm, bnv, l2w, l2b):
    H2 = 2 * D
    full = pl.BlockSpec((D, D), lambda i: (0, 0))
    bias = pl.BlockSpec((1, D), lambda i: (0, 0))
    vec2 = pl.BlockSpec((1, H2), lambda i: (0, 0))
    return pl.pallas_call(
        _mlp_head_body,
        grid=(NBLK,),
        in_specs=[
            pl.BlockSpec((BLK, D), lambda i: (i, 0)),
            pl.BlockSpec((NC, BLK, D), lambda i: (0, i, 0)),
            full, bias, full, bias,
            pl.BlockSpec((1, 1, BLK), lambda i: (i, 0, 0)),
            pl.BlockSpec((D, H2), lambda i: (0, 0)),
            vec2, vec2, vec2, vec2, vec2,
            pl.BlockSpec((H2, D), lambda i: (0, 0)),
            bias,
        ],
        out_specs=pl.BlockSpec((G, D), lambda i: (0, 0)),
        out_shape=jax.ShapeDtypeStruct((G, D), jnp.float32),
        scratch_shapes=[
            pltpu.VMEM((G, D), jnp.float32),
            pltpu.VMEM((G, D), jnp.float32),
        ],
    )(h, agg, wa, ba.reshape(1, D), wb, bb.reshape(1, D), batch3,
      l1w, l1b, bng, bnb, bnm, bnv, l2w, l2b)


def kernel(x, edge_index, batch, w1a, b1a, w1b, b1b, w2a, b2a, w2b, b2b,
           w3a, b3a, w3b, b3b, lin1_w, lin1_b, bn_g, bn_b, bn_rm, bn_rv,
           lin2_w, lin2_b):
    x = x.astype(jnp.float32)
    src = edge_index[0].astype(jnp.int32)
    dst = edge_index[1].astype(jnp.int32)
    cap0 = NS * NCH0 * CHUNK
    src_p = jnp.concatenate([src, jnp.full((E_PAD - E,), N, jnp.int32)])
    dst_p = jnp.concatenate([dst, jnp.zeros((E_PAD - E,), jnp.int32)])

    def tile_layout(v):
        v0 = v[:cap0].reshape(NS, NCH0, 1, CHUNK)
        v1 = v[cap0:].reshape(NS, NCH1, 1, CHUNK)
        v0 = jnp.pad(v0, ((0, 0), (0, NCH - NCH0), (0, 0), (0, 0)),
                     constant_values=N)
        v1 = jnp.pad(v1, ((0, 0), (0, NCH - NCH1), (0, 0), (0, 0)),
                     constant_values=N)
        return jnp.concatenate([v0, v1], axis=0)

    idx4 = jnp.concatenate([tile_layout(src_p),
                            jnp.where(tile_layout(dst_p) == N, 0,
                                      tile_layout(dst_p)),
                            jnp.zeros((NTILES, NCH, 6, CHUNK), jnp.int32)],
                           axis=2)
    zeros = jnp.zeros((SLAB, D), jnp.float32)

    x_pad = jnp.zeros((NPAD, D), jnp.float32).at[:N].set(x)
    batch3 = jnp.concatenate([
        batch.astype(jnp.int32), jnp.full((NPAD - N,), G, jnp.int32)
    ]).reshape(NBLK, 1, BLK)

    h = x_pad
    for wa, ba, wb, bb in ((w1a, b1a, w1b, b1b),
                           (w2a, b2a, w2b, b2b)):
        agg = _make_sc_agg()(h, idx4, zeros)
        h = _mlp(h, agg, wa, ba, wb, bb)

    H2 = 2 * D
    l2w_pad = jnp.zeros((H2, D), jnp.float32).at[:, :NUM_CLASSES].set(lin2_w)
    l2b_pad = jnp.zeros((1, D), jnp.float32).at[0, :NUM_CLASSES].set(lin2_b)
    agg = _make_sc_agg()(h, idx4, zeros)
    out = _mlp_head(h, agg, w3a, b3a, w3b, b3b, batch3,
                    lin1_w, lin1_b.reshape(1, H2),
                    bn_g.reshape(1, H2), bn_b.reshape(1, H2),
                    bn_rm.reshape(1, H2), bn_rv.reshape(1, H2),
                    l2w_pad, l2b_pad)
    return out[:, :NUM_CLASSES]


# speedup vs baseline: 1.1472x; 1.0445x over previous
import functools

import jax
import jax.numpy as jnp
from jax import lax
from jax.experimental import pallas as pl
from jax.experimental.pallas import tpu as pltpu
from jax.experimental.pallas import tpu_sc as plsc

N = 10000
E = 320000
D = 128
G = 128
NUM_CLASSES = 7

NC, NS = 2, 16
NTILES = NC * NS
CHUNK = 112
NSLOT = 3
NISLOT = 6
NCH0 = 168
NCH1 = 12
NCH = max(NCH0, NCH1)
E_PAD = NS * (NCH0 + NCH1) * CHUNK
NPAD = 10240
NACC = 10112
SLAB = NACC // NS

BLK = 256
NBLK = NPAD // BLK


@functools.cache
def _make_sc_agg():
    @functools.partial(
        pl.kernel,
        out_type=jax.ShapeDtypeStruct((NC, NPAD, D), jnp.float32),
        mesh=plsc.VectorSubcoreMesh(
            core_axis_name="c", subcore_axis_name="s",
            num_cores=NC, num_subcores=NS),
        scratch_types=[
            [pltpu.VMEM((8, CHUNK), jnp.int32) for _ in range(NISLOT)],
            [pltpu.VMEM((CHUNK, D), jnp.float32) for _ in range(NSLOT)],
            pltpu.VMEM_SHARED((NACC, D), jnp.float32),
            [pltpu.SemaphoreType.DMA for _ in range(NISLOT)],
            [pltpu.SemaphoreType.DMA for _ in range(NSLOT)],
            [pltpu.SemaphoreType.DMA for _ in range(NSLOT)],
        ],
    )
    def _sc_agg(x_hbm, idx_hbm, out_hbm,
                islots, bufs, acc, isems, gsems, ssems):
        c = lax.axis_index("c")
        s = lax.axis_index("s")
        wid = c * NS + s
        nch = jnp.where(c == 0, NCH0, NCH1)

        def idx_start(j, i):
            pltpu.async_copy(idx_hbm.at[wid, j], islots[i], isems[i])

        def idx_wait(i):
            pltpu.make_async_copy(
                idx_hbm.at[wid, 0], islots[i], isems[i]).wait()

        def gather_start(i, b):
            pltpu.async_copy(x_hbm.at[islots[i].at[0]], bufs[b], gsems[b])

        def gather_wait(i, b):
            pltpu.make_async_copy(
                x_hbm.at[islots[i].at[0]], bufs[b], gsems[b]).wait()

        def scatter_start(i, b):
            pltpu.async_copy(bufs[b], acc.at[islots[i].at[1]], ssems[b],
                             add=True)

        def scatter_wait(i, b):
            pltpu.make_async_copy(
                bufs[b], acc.at[islots[i].at[1]], ssems[b]).wait()

        for i in range(NISLOT - 1):
            idx_start(i, i)

        z16 = jnp.zeros((16,), jnp.float32)

        def zrow(r, carry):
            for k in range(D // 16):
                bufs[0][r, pl.ds(k * 16, 16)] = z16
            return carry

        lax.fori_loop(0, CHUNK, zrow, 0)
        for p in range(SLAB // CHUNK):
            pltpu.sync_copy(bufs[0],
                            acc.at[pl.ds(s * SLAB + p * CHUNK, CHUNK)])
        rem = SLAB - (SLAB // CHUNK) * CHUNK
        if rem:
            pltpu.sync_copy(
                bufs[0].at[pl.ds(0, rem)],
                acc.at[pl.ds(s * SLAB + (SLAB // CHUNK) * CHUNK, rem)])
        plsc.subcore_barrier()
        idx_wait(0)
        gather_start(0, 0)
        idx_wait(1)
        gather_start(1, 1)

        def body(t, carry):
            for b in range(NISLOT):
                j = t * NISLOT + b
                bb = b % NSLOT
                gather_wait(b, bb)
                scatter_start(b, bb)

                @pl.when(j >= 1)
                def _():
                    scatter_wait((b + 5) % NISLOT, (bb + 2) % NSLOT)

                @pl.when(j + 2 < nch)
                def _():
                    idx_wait((b + 2) % NISLOT)
                    gather_start((b + 2) % NISLOT, (bb + 2) % NSLOT)

                @pl.when(j + 5 < nch)
                def _():
                    idx_start(j + 5, (b + 5) % NISLOT)
            return carry

        lax.fori_loop(0, nch // NISLOT, body, 0)
        scatter_wait(NISLOT - 1, NSLOT - 1)
        plsc.subcore_barrier()
        pltpu.sync_copy(acc.at[pl.ds(s * SLAB, SLAB)],
                        out_hbm.at[c, pl.ds(s * SLAB, SLAB)])

    return _sc_agg


def _mlp_body(x_ref, a_ref, wa_ref, ba_ref, wb_ref, bb_ref, o_ref):
    i = pl.program_id(0)
    t = x_ref[...] + a_ref[0] + a_ref[1]
    u = jnp.maximum(
        jnp.dot(t, wa_ref[...], preferred_element_type=jnp.float32)
        + ba_ref[...], 0.0)
    v = jnp.maximum(
        jnp.dot(u, wb_ref[...], preferred_element_type=jnp.float32)
        + bb_ref[...], 0.0)
    rows = i * BLK + lax.broadcasted_iota(jnp.int32, (BLK, D), 0)
    o_ref[...] = jnp.where(rows < N, v, 0.0)


def _mlp(x, a, wa, ba, wb, bb):
    row_spec = pl.BlockSpec((BLK, D), lambda i: (i, 0))
    agg_spec = pl.BlockSpec((NC, BLK, D), lambda i: (0, i, 0))
    full = pl.BlockSpec((D, D), lambda i: (0, 0))
    bias = pl.BlockSpec((1, D), lambda i: (0, 0))
    return pl.pallas_call(
        _mlp_body,
        grid=(NBLK,),
        in_specs=[row_spec, agg_spec, full, bias, full, bias],
        out_specs=row_spec,
        out_shape=jax.ShapeDtypeStruct((NPAD, D), jnp.float32),
    )(x, a, wa, ba.reshape(1, D), wb, bb.reshape(1, D))


def _mlp_head_body(x_ref, a_ref, wa_ref, ba_ref, wb_ref, bb_ref, b_ref,
                   l1w_ref, l1b_ref, bng_ref, bnb_ref, bnm_ref, bnv_ref,
                   l2w_ref, l2b_ref, o_ref, sums, cnts):
    i = pl.program_id(0)

    @pl.when(i == 0)
    def _():
        sums[...] = jnp.zeros_like(sums)
        cnts[...] = jnp.zeros_like(cnts)

    t = x_ref[...] + a_ref[0] + a_ref[1]
    u = jnp.maximum(
        jnp.dot(t, wa_ref[...], preferred_element_type=jnp.float32)
        + ba_ref[...], 0.0)
    v = jnp.maximum(
        jnp.dot(u, wb_ref[...], preferred_element_type=jnp.float32)
        + bb_ref[...], 0.0)
    rows = i * BLK + lax.broadcasted_iota(jnp.int32, (BLK, D), 0)
    v = jnp.where(rows < N, v, 0.0)

    b = b_ref[0, 0, :]
    gids = lax.broadcasted_iota(jnp.int32, (G, BLK), 0)
    mask = (b[None, :] == gids).astype(jnp.float32)
    sums[...] += jnp.dot(mask, v, preferred_element_type=jnp.float32)
    cnts[...] += jnp.broadcast_to(
        jnp.sum(mask, axis=1, keepdims=True), (G, D))

    @pl.when(i == NBLK - 1)
    def _():
        pooled = sums[...] / jnp.maximum(cnts[...], 1.0)
        g = jnp.dot(pooled, l1w_ref[...],
                    preferred_element_type=jnp.float32) + l1b_ref[...]
        g = (g - bnm_ref[...]) * jax.lax.rsqrt(bnv_ref[...] + 1e-5)
        g = g * bng_ref[...] + bnb_ref[...]
        g = jnp.maximum(g, 0.0)
        o_ref[...] = jnp.dot(g, l2w_ref[...],
                             preferred_element_type=jnp.float32) + l2b_ref[...]


def _mlp_head(h, agg, wa, ba, wb, bb, batch3,
              l1w, l1b, bng, bnb, bnm, bnv, l2w, l2b):
    H2 = 2 * D
    full = pl.BlockSpec((D, D), lambda i: (0, 0))
    bias = pl.BlockSpec((1, D), lambda i: (0, 0))
    vec2 = pl.BlockSpec((1, H2), lambda i: (0, 0))
    return pl.pallas_call(
        _mlp_head_body,
        grid=(NBLK,),
        in_specs=[
            pl.BlockSpec((BLK, D), lambda i: (i, 0)),
            pl.BlockSpec((NC, BLK, D), lambda i: (0, i, 0)),
            full, bias, full, bias,
            pl.BlockSpec((1, 1, BLK), lambda i: (i, 0, 0)),
            pl.BlockSpec((D, H2), lambda i: (0, 0)),
            vec2, vec2, vec2, vec2, vec2,
            pl.BlockSpec((H2, D), lambda i: (0, 0)),
            bias,
        ],
        out_specs=pl.BlockSpec((G, D), lambda i: (0, 0)),
        out_shape=jax.ShapeDtypeStruct((G, D), jnp.float32),
        scratch_shapes=[
            pltpu.VMEM((G, D), jnp.float32),
            pltpu.VMEM((G, D), jnp.float32),
        ],
    )(h, agg, wa, ba.reshape(1, D), wb, bb.reshape(1, D), batch3,
      l1w, l1b, bng, bnb, bnm, bnv, l2w, l2b)


def kernel(x, edge_index, batch, w1a, b1a, w1b, b1b, w2a, b2a, w2b, b2b,
           w3a, b3a, w3b, b3b, lin1_w, lin1_b, bn_g, bn_b, bn_rm, bn_rv,
           lin2_w, lin2_b):
    x = x.astype(jnp.float32)
    src = edge_index[0].astype(jnp.int32)
    dst = edge_index[1].astype(jnp.int32)
    cap0 = NS * NCH0 * CHUNK
    src_p = jnp.concatenate([src, jnp.full((E_PAD - E,), N, jnp.int32)])
    dst_p = jnp.concatenate([dst, jnp.zeros((E_PAD - E,), jnp.int32)])

    def tile_layout(v):
        v0 = v[:cap0].reshape(NS, NCH0, 1, CHUNK)
        v1 = v[cap0:].reshape(NS, NCH1, 1, CHUNK)
        v0 = jnp.pad(v0, ((0, 0), (0, NCH - NCH0), (0, 0), (0, 0)),
                     constant_values=N)
        v1 = jnp.pad(v1, ((0, 0), (0, NCH - NCH1), (0, 0), (0, 0)),
                     constant_values=N)
        return jnp.concatenate([v0, v1], axis=0)

    idx4 = jnp.concatenate([tile_layout(src_p),
                            jnp.where(tile_layout(dst_p) == N, 0,
                                      tile_layout(dst_p)),
                            jnp.zeros((NTILES, NCH, 6, CHUNK), jnp.int32)],
                           axis=2)

    x_pad = jnp.zeros((NPAD, D), jnp.float32).at[:N].set(x)
    batch3 = jnp.concatenate([
        batch.astype(jnp.int32), jnp.full((NPAD - N,), G, jnp.int32)
    ]).reshape(NBLK, 1, BLK)

    h = x_pad
    for wa, ba, wb, bb in ((w1a, b1a, w1b, b1b),
                           (w2a, b2a, w2b, b2b)):
        agg = _make_sc_agg()(h, idx4)
        h = _mlp(h, agg, wa, ba, wb, bb)

    H2 = 2 * D
    l2w_pad = jnp.zeros((H2, D), jnp.float32).at[:, :NUM_CLASSES].set(lin2_w)
    l2b_pad = jnp.zeros((1, D), jnp.float32).at[0, :NUM_CLASSES].set(lin2_b)
    agg = _make_sc_agg()(h, idx4)
    out = _mlp_head(h, agg, w3a, b3a, w3b, b3b, batch3,
                    lin1_w, lin1_b.reshape(1, H2),
                    bn_g.reshape(1, H2), bn_b.reshape(1, H2),
                    bn_rm.reshape(1, H2), bn_rv.reshape(1, H2),
                    l2w_pad, l2b_pad)
    return out[:, :NUM_CLASSES]
